# Initial kernel scaffold; baseline (speedup 1.0000x reference)
import jax
import jax.numpy as jnp
from jax.experimental import pallas as pl


def kernel(dispatched, gate_up_proj, gate_up_proj_bias, down_proj, down_proj_bias, sparsity_remap):
    raise NotImplementedError("write your pallas kernel here")


# speedup vs baseline: 2.2231x; 2.2231x over previous
import jax
import jax.numpy as jnp
from jax.experimental import pallas as pl

_A, _B, _E, _M, _H = 4, 4, 8, 128, 1024
_I = 1024
_G = _A * _B
_S = 2048


def _fused_expert_body(x_ref, guw_ref, gub_ref, dw_ref, db_ref, o_ref):
    x = x_ref[0, 0].astype(jnp.bfloat16)
    guw = guw_ref[0].astype(jnp.bfloat16)
    gu = jnp.dot(x, guw, preferred_element_type=jnp.float32)
    gu = gu + gub_ref[0]
    gate = gu[:, :_I]
    up = gu[:, _I:]
    act = (gate * jax.nn.sigmoid(gate)) * up
    act = act.astype(jnp.bfloat16)
    dw = dw_ref[0].astype(jnp.bfloat16)
    out = jnp.dot(act, dw, preferred_element_type=jnp.float32)
    o_ref[...] = out + db_ref[0]


def kernel(dispatched, gate_up_proj, gate_up_proj_bias, down_proj, down_proj_bias, sparsity_remap):
    del sparsity_remap

    x = dispatched.reshape(_G, _E, _M, _H)
    guw = gate_up_proj.reshape(_E, _H, _I, 2).transpose(0, 1, 3, 2).reshape(_E, _H, 2 * _I)
    gub = gate_up_proj_bias.reshape(_E, _I, 2).transpose(0, 2, 1).reshape(_E, 1, 2 * _I)
    db = down_proj_bias.reshape(_E, 1, _H)

    out2d = pl.pallas_call(
        _fused_expert_body,
        grid=(_E, _G),
        in_specs=[
            pl.BlockSpec((1, 1, _M, _H), lambda e, g: (g, e, 0, 0)),
            pl.BlockSpec((1, _H, 2 * _I), lambda e, g: (e, 0, 0)),
            pl.BlockSpec((1, 1, 2 * _I), lambda e, g: (e, 0, 0)),
            pl.BlockSpec((1, _I, _H), lambda e, g: (e, 0, 0)),
            pl.BlockSpec((1, 1, _H), lambda e, g: (e, 0, 0)),
        ],
        out_specs=pl.BlockSpec((_M, _H), lambda e, g: (g, e)),
        out_shape=jax.ShapeDtypeStruct((_S, _E * _H), jnp.float32),
    )(x, guw, gub, down_proj, db)

    return out2d.reshape(1, _S, _E, _H)


# speedup vs baseline: 2.3822x; 1.0715x over previous
import jax
import jax.numpy as jnp
from jax.experimental import pallas as pl

_A, _B, _E, _M, _H = 4, 4, 8, 128, 1024
_I = 1024
_G = _A * _B
_S = 2048


def _fused_expert_body(x_ref, guw_ref, gub_ref, dw_ref, db_ref, o_ref):
    x = x_ref[0, 0].astype(jnp.bfloat16)
    guw = guw_ref[0]
    gu = jnp.dot(x, guw, preferred_element_type=jnp.float32)
    gu = gu + gub_ref[0]
    gate = gu[:, :_I]
    up = gu[:, _I:]
    act = (gate * jax.nn.sigmoid(gate)) * up
    act = act.astype(jnp.bfloat16)
    out = jnp.dot(act, dw_ref[0], preferred_element_type=jnp.float32)
    o_ref[...] = out + db_ref[0]


def kernel(dispatched, gate_up_proj, gate_up_proj_bias, down_proj, down_proj_bias, sparsity_remap):
    del sparsity_remap

    x = dispatched.reshape(_G, _E, _M, _H)
    guw = (gate_up_proj.reshape(_E, _H, _I, 2).transpose(0, 1, 3, 2)
           .reshape(_E, _H, 2 * _I).astype(jnp.bfloat16))
    dw = down_proj.astype(jnp.bfloat16)
    gub = gate_up_proj_bias.reshape(_E, _I, 2).transpose(0, 2, 1).reshape(_E, 1, 2 * _I)
    db = down_proj_bias.reshape(_E, 1, _H)

    out2d = pl.pallas_call(
        _fused_expert_body,
        grid=(_E, _G),
        in_specs=[
            pl.BlockSpec((1, 1, _M, _H), lambda e, g: (g, e, 0, 0)),
            pl.BlockSpec((1, _H, 2 * _I), lambda e, g: (e, 0, 0)),
            pl.BlockSpec((1, 1, 2 * _I), lambda e, g: (e, 0, 0)),
            pl.BlockSpec((1, _I, _H), lambda e, g: (e, 0, 0)),
            pl.BlockSpec((1, 1, _H), lambda e, g: (e, 0, 0)),
        ],
        out_specs=pl.BlockSpec((_M, _H), lambda e, g: (g, e)),
        out_shape=jax.ShapeDtypeStruct((_S, _E * _H), jnp.float32),
    )(x, guw, gub, dw, db)

    return out2d.reshape(1, _S, _E, _H)


# speedup vs baseline: 2.6159x; 1.0981x over previous
import jax
import jax.numpy as jnp
from jax.experimental import pallas as pl
from jax.experimental.pallas import tpu as pltpu

_A, _B, _E, _M, _H = 4, 4, 8, 128, 1024
_I = 1024
_G = _A * _B
_S = 2048
_GP = 2
_MM = _GP * _M


def _fused_expert_body(x_ref, guw_ref, gub_ref, dw_ref, db_ref, o_ref):
    x = x_ref[:, 0].reshape(_MM, _H).astype(jnp.bfloat16)
    guw = guw_ref[0]
    gu = jnp.dot(x, guw, preferred_element_type=jnp.float32)
    gu = gu + gub_ref[0]
    gate = gu[:, :_I]
    up = gu[:, _I:]
    act = (gate * jax.nn.sigmoid(gate)) * up
    act = act.astype(jnp.bfloat16)
    out = jnp.dot(act, dw_ref[0], preferred_element_type=jnp.float32)
    o_ref[...] = out + db_ref[0]


def kernel(dispatched, gate_up_proj, gate_up_proj_bias, down_proj, down_proj_bias, sparsity_remap):
    del sparsity_remap

    x = dispatched.reshape(_G, _E, _M, _H)
    guw = (gate_up_proj.reshape(_E, _H, _I, 2).transpose(0, 1, 3, 2)
           .reshape(_E, _H, 2 * _I).astype(jnp.bfloat16))
    dw = down_proj.astype(jnp.bfloat16)
    gub = gate_up_proj_bias.reshape(_E, _I, 2).transpose(0, 2, 1).reshape(_E, 1, 2 * _I)
    db = down_proj_bias.reshape(_E, 1, _H)

    out2d = pl.pallas_call(
        _fused_expert_body,
        grid=(_E, _G // _GP),
        in_specs=[
            pl.BlockSpec((_GP, 1, _M, _H), lambda e, g: (g, e, 0, 0)),
            pl.BlockSpec((1, _H, 2 * _I), lambda e, g: (e, 0, 0)),
            pl.BlockSpec((1, 1, 2 * _I), lambda e, g: (e, 0, 0)),
            pl.BlockSpec((1, _I, _H), lambda e, g: (e, 0, 0)),
            pl.BlockSpec((1, 1, _H), lambda e, g: (e, 0, 0)),
        ],
        out_specs=pl.BlockSpec((_MM, _H), lambda e, g: (g, e)),
        out_shape=jax.ShapeDtypeStruct((_S, _E * _H), jnp.float32),
        compiler_params=pltpu.CompilerParams(
            dimension_semantics=("arbitrary", "arbitrary"),
        ),
    )(x, guw, gub, dw, db)

    return out2d.reshape(1, _S, _E, _H)


# speedup vs baseline: 2.6521x; 1.0138x over previous
import jax
import jax.numpy as jnp
from jax.experimental import pallas as pl
from jax.experimental.pallas import tpu as pltpu

_A, _B, _E, _M, _H = 4, 4, 8, 128, 1024
_I = 1024
_G = _A * _B
_S = 2048
_GP = 2
_MM = _GP * _M


def _fused_expert_body(x_ref, guw_ref, gub_ref, dw_ref, db_ref, o_ref, dw_s):
    @pl.when(pl.program_id(1) == 0)
    def _prep():
        dw_s[...] = dw_ref[0].astype(jnp.bfloat16)

    x = x_ref[:, 0].reshape(_MM, _H).astype(jnp.bfloat16)
    gu = jnp.dot(x, guw_ref[0], preferred_element_type=jnp.float32)
    gu = gu + gub_ref[0]
    gate = gu[:, :_I]
    up = gu[:, _I:]
    act = (gate * jax.nn.sigmoid(gate)) * up
    act = act.astype(jnp.bfloat16)
    out = jnp.dot(act, dw_s[...], preferred_element_type=jnp.float32)
    o_ref[...] = out + db_ref[0]


def kernel(dispatched, gate_up_proj, gate_up_proj_bias, down_proj, down_proj_bias, sparsity_remap):
    del sparsity_remap

    x = dispatched.reshape(_G, _E, _M, _H)
    guw = (gate_up_proj.reshape(_E, _H, _I, 2).transpose(0, 1, 3, 2)
           .reshape(_E, _H, 2 * _I).astype(jnp.bfloat16))
    gub = gate_up_proj_bias.reshape(_E, _I, 2).transpose(0, 2, 1).reshape(_E, 1, 2 * _I)
    db = down_proj_bias.reshape(_E, 1, _H)

    out2d = pl.pallas_call(
        _fused_expert_body,
        grid=(_E, _G // _GP),
        in_specs=[
            pl.BlockSpec((_GP, 1, _M, _H), lambda e, g: (g, e, 0, 0)),
            pl.BlockSpec((1, _H, 2 * _I), lambda e, g: (e, 0, 0)),
            pl.BlockSpec((1, 1, 2 * _I), lambda e, g: (e, 0, 0)),
            pl.BlockSpec((1, _I, _H), lambda e, g: (e, 0, 0)),
            pl.BlockSpec((1, 1, _H), lambda e, g: (e, 0, 0)),
        ],
        out_specs=pl.BlockSpec((_MM, _H), lambda e, g: (g, e)),
        out_shape=jax.ShapeDtypeStruct((_S, _E * _H), jnp.float32),
        scratch_shapes=[
            pltpu.VMEM((_I, _H), jnp.bfloat16),
        ],
        compiler_params=pltpu.CompilerParams(
            dimension_semantics=("arbitrary", "arbitrary"),
        ),
    )(x, guw, gub, down_proj, db)

    return out2d.reshape(1, _S, _E, _H)


# speedup vs baseline: 3.2280x; 1.2172x over previous
import jax
import jax.numpy as jnp
from jax.experimental import pallas as pl
from jax.experimental.pallas import tpu as pltpu

_A, _B, _E, _M, _H = 4, 4, 8, 128, 1024
_I = 1024
_G = _A * _B
_S = 2048
_GP = 2
_MM = _GP * _M


def _fused_expert_body(x_ref, guw_ref, gub_ref, dw_ref, db_ref, o_ref,
                       psel_s, guw_s, dw_s, gub_s):
    @pl.when((pl.program_id(0) == 0) & (pl.program_id(1) == 0))
    def _build_psel():
        n = jax.lax.broadcasted_iota(jnp.int32, (2 * _I, 2 * _I), 0)
        i = jax.lax.broadcasted_iota(jnp.int32, (2 * _I, 2 * _I), 1)
        src = jnp.where(i < _I, 2 * i, 2 * (i - _I) + 1)
        psel_s[...] = (n == src).astype(jnp.bfloat16)

    @pl.when(pl.program_id(1) == 0)
    def _prep():
        w = guw_ref[0].astype(jnp.bfloat16)
        guw_s[...] = jnp.dot(w, psel_s[...],
                             preferred_element_type=jnp.float32).astype(jnp.bfloat16)
        dw_s[...] = dw_ref[0].astype(jnp.bfloat16)
        b = gub_ref[0].astype(jnp.bfloat16)
        gub_s[...] = jnp.dot(b, psel_s[...], preferred_element_type=jnp.float32)

    x = x_ref[:, 0].reshape(_MM, _H).astype(jnp.bfloat16)
    gu = jnp.dot(x, guw_s[...], preferred_element_type=jnp.float32)
    gu = gu + gub_s[...]
    gate = gu[:, :_I]
    up = gu[:, _I:]
    act = (gate * jax.nn.sigmoid(gate)) * up
    act = act.astype(jnp.bfloat16)
    out = jnp.dot(act, dw_s[...], preferred_element_type=jnp.float32)
    o_ref[...] = out + db_ref[0]


def kernel(dispatched, gate_up_proj, gate_up_proj_bias, down_proj, down_proj_bias, sparsity_remap):
    del sparsity_remap

    x = dispatched.reshape(_G, _E, _M, _H)
    gub = gate_up_proj_bias.reshape(_E, 1, 2 * _I)
    db = down_proj_bias.reshape(_E, 1, _H)

    out2d = pl.pallas_call(
        _fused_expert_body,
        grid=(_E, _G // _GP),
        in_specs=[
            pl.BlockSpec((_GP, 1, _M, _H), lambda e, g: (g, e, 0, 0)),
            pl.BlockSpec((1, _H, 2 * _I), lambda e, g: (e, 0, 0)),
            pl.BlockSpec((1, 1, 2 * _I), lambda e, g: (e, 0, 0)),
            pl.BlockSpec((1, _I, _H), lambda e, g: (e, 0, 0)),
            pl.BlockSpec((1, 1, _H), lambda e, g: (e, 0, 0)),
        ],
        out_specs=pl.BlockSpec((_MM, _H), lambda e, g: (g, e)),
        out_shape=jax.ShapeDtypeStruct((_S, _E * _H), jnp.float32),
        scratch_shapes=[
            pltpu.VMEM((2 * _I, 2 * _I), jnp.bfloat16),
            pltpu.VMEM((_H, 2 * _I), jnp.bfloat16),
            pltpu.VMEM((_I, _H), jnp.bfloat16),
            pltpu.VMEM((1, 2 * _I), jnp.float32),
        ],
        compiler_params=pltpu.CompilerParams(
            dimension_semantics=("arbitrary", "arbitrary"),
        ),
    )(x, gate_up_proj, gub, down_proj, db)

    return out2d.reshape(1, _S, _E, _H)


# speedup vs baseline: 3.4217x; 1.0600x over previous
import jax
import jax.numpy as jnp
from jax.experimental import pallas as pl
from jax.experimental.pallas import tpu as pltpu

_A, _B, _E, _M, _H = 4, 4, 8, 128, 1024
_I = 1024
_G = _A * _B
_S = 2048
_GP = 4
_MM = _GP * _M


def _fused_expert_body(x_ref, guw_ref, gub_ref, dw_ref, db_ref, o_ref,
                       psel_s, guw_s, dw_s, gub_s):
    @pl.when((pl.program_id(0) == 0) & (pl.program_id(1) == 0))
    def _build_psel():
        n = jax.lax.broadcasted_iota(jnp.int32, (2 * _I, 2 * _I), 0)
        i = jax.lax.broadcasted_iota(jnp.int32, (2 * _I, 2 * _I), 1)
        src = jnp.where(i < _I, 2 * i, 2 * (i - _I) + 1)
        psel_s[...] = (n == src).astype(jnp.bfloat16)

    @pl.when(pl.program_id(1) == 0)
    def _prep():
        w = guw_ref[0].astype(jnp.bfloat16)
        guw_s[...] = jnp.dot(w, psel_s[...],
                             preferred_element_type=jnp.float32).astype(jnp.bfloat16)
        dw_s[...] = dw_ref[0].astype(jnp.bfloat16)
        b = gub_ref[0].astype(jnp.bfloat16)
        gub_s[...] = jnp.dot(b, psel_s[...], preferred_element_type=jnp.float32)

    x = x_ref[:, 0].reshape(_MM, _H).astype(jnp.bfloat16)
    gu = jnp.dot(x, guw_s[...], preferred_element_type=jnp.float32)
    gu = gu + gub_s[...]
    gate = gu[:, :_I]
    up = gu[:, _I:]
    act = (gate * jax.nn.sigmoid(gate)) * up
    act = act.astype(jnp.bfloat16)
    out = jnp.dot(act, dw_s[...], preferred_element_type=jnp.float32)
    o_ref[...] = out + db_ref[0]


def kernel(dispatched, gate_up_proj, gate_up_proj_bias, down_proj, down_proj_bias, sparsity_remap):
    del sparsity_remap

    x = dispatched.reshape(_G, _E, _M, _H)
    gub = gate_up_proj_bias.reshape(_E, 1, 2 * _I)
    db = down_proj_bias.reshape(_E, 1, _H)

    out2d = pl.pallas_call(
        _fused_expert_body,
        grid=(_E, _G // _GP),
        in_specs=[
            pl.BlockSpec((_GP, 1, _M, _H), lambda e, g: (g, e, 0, 0)),
            pl.BlockSpec((1, _H, 2 * _I), lambda e, g: (e, 0, 0)),
            pl.BlockSpec((1, 1, 2 * _I), lambda e, g: (e, 0, 0)),
            pl.BlockSpec((1, _I, _H), lambda e, g: (e, 0, 0)),
            pl.BlockSpec((1, 1, _H), lambda e, g: (e, 0, 0)),
        ],
        out_specs=pl.BlockSpec((_MM, _H), lambda e, g: (g, e)),
        out_shape=jax.ShapeDtypeStruct((_S, _E * _H), jnp.float32),
        scratch_shapes=[
            pltpu.VMEM((2 * _I, 2 * _I), jnp.bfloat16),
            pltpu.VMEM((_H, 2 * _I), jnp.bfloat16),
            pltpu.VMEM((_I, _H), jnp.bfloat16),
            pltpu.VMEM((1, 2 * _I), jnp.float32),
        ],
        compiler_params=pltpu.CompilerParams(
            dimension_semantics=("arbitrary", "arbitrary"),
        ),
    )(x, gate_up_proj, gub, down_proj, db)

    return out2d.reshape(1, _S, _E, _H)
